# baseline (speedup 1.0000x reference)
import jax
import jax.numpy as jnp
from jax import lax
from jax.experimental import pallas as pl
from jax.experimental.pallas import tpu as pltpu

_B, _N, _D = 16, 4096, 256
_K = 10
_T = 128
_W = 24


def _topk_rounds(v, base):
    cols = []
    for _ in range(_K):
        mx = jnp.max(v, axis=1)
        cols.append(jnp.maximum(mx + base, 0))
        v = jnp.where(v == mx[:, None], -1, v)
    cols.reverse()
    return jnp.stack(cols, axis=1)


def _body(x_hbm, mask_hbm, o_ref, win_v, tail_v, mfull_v, sem, semt, sem2):
    m8 = mask_hbm
    wspec = _N - _W
    spec_cp = pltpu.make_async_copy(
        x_hbm.at[:, pl.ds(wspec, _W), :], win_v, sem
    )
    spec_cp.start()
    tail_cp = pltpu.make_async_copy(
        m8.at[:, pl.ds(_N - _T, _T)], tail_v, semt
    )
    tail_cp.start()
    tail_cp.wait()

    mt = tail_v[...].astype(jnp.int32)
    all_dense = jnp.min(jnp.sum(mt, axis=1)) >= _K

    @pl.when(jnp.logical_not(all_dense))
    def _():
        cp = pltpu.make_async_copy(m8, mfull_v, sem2)
        cp.start()
        cp.wait()

    def tail_path(_):
        vt = lax.broadcasted_iota(jnp.int32, (_B, _T), 1) * mt
        return _topk_rounds(vt, _N - _T)

    def full_path(_):
        m = mfull_v[...].astype(jnp.int32)
        v = lax.broadcasted_iota(jnp.int32, (_B, _N), 1) * m
        return _topk_rounds(v, 0)

    slots = lax.cond(all_dense, tail_path, full_path, 0)

    bi = lax.broadcasted_iota(jnp.int32, (_B, _K), 0)
    ci = lax.broadcasted_iota(jnp.int32, (_B, _K), 1)
    base_v = slots[:, :1]
    base = jnp.max(base_v)
    fast = (
        (jnp.sum(jnp.where(slots == base_v + ci, 1, 0)) == _B * _K)
        & (jnp.min(base_v) == base)
        & (base >= wspec)
    )
    spec_cp.wait()

    @pl.when(fast)
    def _():
        rolled = pltpu.roll(win_v[...], (_W - (base - wspec)) % _W, 1)
        for c in range(_K):
            o_ref[:, pl.ds(c * _D, _D)] = rolled[:, c, :]

    @pl.when(jnp.logical_not(fast))
    def _():
        bi2 = lax.broadcasted_iota(jnp.int32, (_B, _K * _D), 0)
        cd2 = lax.broadcasted_iota(jnp.int32, (_B, _K * _D), 1) // _D

        def one(g, acc):
            i = g // _K
            c = g % _K
            s = jnp.sum(jnp.where((bi == i) & (ci == c), slots, 0))
            w = (s // 8) * 8
            cp = pltpu.make_async_copy(
                x_hbm.at[pl.ds(i, 1), pl.ds(w, 8), :],
                win_v.at[pl.ds(0, 1), pl.ds(0, 8)],
                sem,
            )
            cp.start()
            cp.wait()
            row = pltpu.roll(win_v[0, :8, :], (8 - (s - w)) % 8, 0)[:1]
            rowt = jnp.concatenate([row] * _K, axis=1)
            return jnp.where((bi2 == i) & (cd2 == c), rowt, acc)

        acc = lax.fori_loop(
            0, _B * _K, one, jnp.zeros((_B, _K * _D), jnp.float32)
        )
        o_ref[...] = acc


def kernel(x, layer_layouts, node_mask):
    del layer_layouts

    out = pl.pallas_call(
        _body,
        grid=(1,),
        in_specs=[
            pl.BlockSpec(memory_space=pl.ANY),
            pl.BlockSpec(memory_space=pl.ANY),
        ],
        out_specs=pl.BlockSpec((_B, _K * _D), lambda i: (0, 0)),
        out_shape=jax.ShapeDtypeStruct((_B, _K * _D), jnp.float32),
        scratch_shapes=[
            pltpu.VMEM((_B, _W, _D), jnp.float32),
            pltpu.VMEM((_B, _T), jnp.int8),
            pltpu.VMEM((_B, _N), jnp.int8),
            pltpu.SemaphoreType.DMA,
            pltpu.SemaphoreType.DMA,
            pltpu.SemaphoreType.DMA,
        ],
    )(x, node_mask.view(jnp.int8))
    return out


# speedup vs baseline: 1.0301x; 1.0301x over previous
import jax
import jax.numpy as jnp
from jax import lax
from jax.experimental import pallas as pl
from jax.experimental.pallas import tpu as pltpu

_B, _N, _D = 16, 4096, 256
_K = 10
_T = 128
_W = 24


def _topk_rounds(v, base):
    cols = []
    for _ in range(_K):
        mx = jnp.max(v, axis=1)
        cols.append(jnp.maximum(mx + base, 0))
        v = jnp.where(v == mx[:, None], -1, v)
    cols.reverse()
    return jnp.stack(cols, axis=1)


def _body(tail_ref, x_hbm, mask_hbm, o_ref, win_v, mfull_v, sem, sem2):
    wspec = _N - _W
    spec_cp = pltpu.make_async_copy(
        x_hbm.at[:, pl.ds(wspec, _W), :], win_v, sem
    )
    spec_cp.start()

    mt = tail_ref[...].astype(jnp.int32)
    all_dense = jnp.min(jnp.sum(mt, axis=1)) >= _K

    @pl.when(jnp.logical_not(all_dense))
    def _():
        cp = pltpu.make_async_copy(mask_hbm, mfull_v, sem2)
        cp.start()
        cp.wait()

    def tail_path(_):
        vt = lax.broadcasted_iota(jnp.int32, (_B, _T), 1) * mt
        return _topk_rounds(vt, _N - _T)

    def full_path(_):
        m = mfull_v[...].astype(jnp.int32)
        v = lax.broadcasted_iota(jnp.int32, (_B, _N), 1) * m
        return _topk_rounds(v, 0)

    slots = lax.cond(all_dense, tail_path, full_path, 0)

    bi = lax.broadcasted_iota(jnp.int32, (_B, _K), 0)
    ci = lax.broadcasted_iota(jnp.int32, (_B, _K), 1)
    base_v = slots[:, :1]
    base = jnp.max(base_v)
    fast = (
        (jnp.sum(jnp.where(slots == base_v + ci, 1, 0)) == _B * _K)
        & (jnp.min(base_v) == base)
        & (base >= wspec)
    )
    spec_cp.wait()

    @pl.when(fast)
    def _():
        rolled = pltpu.roll(win_v[...], (_W - (base - wspec)) % _W, 1)
        for c in range(_K):
            o_ref[:, pl.ds(c * _D, _D)] = rolled[:, c, :]

    @pl.when(jnp.logical_not(fast))
    def _():
        bi2 = lax.broadcasted_iota(jnp.int32, (_B, _K * _D), 0)
        cd2 = lax.broadcasted_iota(jnp.int32, (_B, _K * _D), 1) // _D

        def one(g, acc):
            i = g // _K
            c = g % _K
            s = jnp.sum(jnp.where((bi == i) & (ci == c), slots, 0))
            w = (s // 8) * 8
            cp = pltpu.make_async_copy(
                x_hbm.at[pl.ds(i, 1), pl.ds(w, 8), :],
                win_v.at[pl.ds(0, 1), pl.ds(0, 8)],
                sem,
            )
            cp.start()
            cp.wait()
            row = pltpu.roll(win_v[0, :8, :], (8 - (s - w)) % 8, 0)[:1]
            rowt = jnp.concatenate([row] * _K, axis=1)
            return jnp.where((bi2 == i) & (cd2 == c), rowt, acc)

        acc = lax.fori_loop(
            0, _B * _K, one, jnp.zeros((_B, _K * _D), jnp.float32)
        )
        o_ref[...] = acc


def kernel(x, layer_layouts, node_mask):
    del layer_layouts
    mask_i8 = node_mask.view(jnp.int8)

    out = pl.pallas_call(
        _body,
        grid=(1,),
        in_specs=[
            pl.BlockSpec((_B, _T), lambda i: (0, (_N // _T) - 1)),
            pl.BlockSpec(memory_space=pl.ANY),
            pl.BlockSpec(memory_space=pl.ANY),
        ],
        out_specs=pl.BlockSpec((_B, _K * _D), lambda i: (0, 0)),
        out_shape=jax.ShapeDtypeStruct((_B, _K * _D), jnp.float32),
        scratch_shapes=[
            pltpu.VMEM((_B, _W, _D), jnp.float32),
            pltpu.VMEM((_B, _N), jnp.int8),
            pltpu.SemaphoreType.DMA,
            pltpu.SemaphoreType.DMA,
        ],
    )(mask_i8, x, mask_i8)
    return out
